# Initial kernel scaffold; baseline (speedup 1.0000x reference)
import jax
import jax.numpy as jnp
from jax.experimental import pallas as pl


def kernel(scalars, edge_index, edge_len, W1, b1, W2, b2):
    raise NotImplementedError("write your pallas kernel here")


# speedup vs baseline: 2.4354x; 2.4354x over previous
import functools

import jax
import jax.numpy as jnp
from jax import lax
from jax.experimental import pallas as pl
from jax.experimental.pallas import tpu as pltpu
from jax.experimental.pallas import tpu_sc as plsc


def _precompute_body(scalars_ref, wa_ref, wb_ref, pa_ref, pb_ref):
    s = scalars_ref[...]
    pa_ref[...] = jnp.dot(s, wa_ref[...], preferred_element_type=jnp.float32)
    pb_ref[...] = jnp.dot(s, wb_ref[...], preferred_element_type=jnp.float32)


def _precompute(scalars, wa, wb):
    n, _ = scalars.shape
    h = wa.shape[1]
    return pl.pallas_call(
        _precompute_body,
        out_shape=[
            jax.ShapeDtypeStruct((n, h), jnp.float32),
            jax.ShapeDtypeStruct((n, h), jnp.float32),
        ],
    )(scalars, wa, wb)


def _sc_gather(pa, pb, snd, rcv, chunk=400):
    n, h = pa.shape
    e = snd.shape[0]
    info = plsc.get_sparse_core_info()
    nc, ns = info.num_cores, info.num_subcores
    nw = nc * ns
    assert e % nw == 0
    epw = e // nw
    assert epw % chunk == 0 and chunk % 8 == 0
    steps = epw // chunk
    mesh = plsc.VectorSubcoreMesh(core_axis_name="c", subcore_axis_name="s")

    @functools.partial(
        pl.kernel,
        mesh=mesh,
        compiler_params=pltpu.CompilerParams(use_tc_tiling_on_sc=False),
        out_type=(
            jax.ShapeDtypeStruct((e, h), jnp.float32),
            jax.ShapeDtypeStruct((e, h), jnp.float32),
        ),
        scratch_types=[
            pltpu.VMEM((chunk,), jnp.int32),
            pltpu.VMEM((chunk,), jnp.int32),
            pltpu.VMEM((chunk, h), jnp.float32),
            pltpu.VMEM((chunk, h), jnp.float32),
            pltpu.SemaphoreType.DMA,
            pltpu.SemaphoreType.DMA,
        ],
    )
    def k(pa_hbm, pb_hbm, snd_hbm, rcv_hbm, ga_hbm, gb_hbm,
          idxa_v, idxb_v, bufa_v, bufb_v, sema, semb):
        wid = lax.axis_index("s") * nc + lax.axis_index("c")
        base0 = wid * epw

        def body(kk, carry):
            base = base0 + kk * chunk
            pltpu.sync_copy(snd_hbm.at[pl.ds(base, chunk)], idxa_v)
            pltpu.sync_copy(rcv_hbm.at[pl.ds(base, chunk)], idxb_v)
            cpa = pltpu.async_copy(pa_hbm.at[idxa_v], bufa_v, sema)
            cpb = pltpu.async_copy(pb_hbm.at[idxb_v], bufb_v, semb)
            cpa.wait()
            cpb.wait()
            pltpu.sync_copy(bufa_v, ga_hbm.at[pl.ds(base, chunk)])
            pltpu.sync_copy(bufb_v, gb_hbm.at[pl.ds(base, chunk)])
            return carry

        lax.fori_loop(0, steps, body, 0)

    return k(pa, pb, snd, rcv)


def _mlp_body(ga_ref, gb_ref, len_ref, w1c_ref, b1_ref, w2_ref, b2_ref,
              out_ref):
    x = ga_ref[...] + gb_ref[...] + len_ref[...] * w1c_ref[...] + b1_ref[...]
    h = x * jax.nn.sigmoid(x)
    out_ref[...] = (
        jnp.dot(h, w2_ref[...], preferred_element_type=jnp.float32)
        + b2_ref[...]
    )


def _edge_mlp(ga, gb, len2d, w1c, b1, w2, b2, block=3200):
    e, h = ga.shape
    assert e % block == 0
    grid = (e // block,)
    row = lambda i: (i, 0)
    full = lambda i: (0, 0)
    return pl.pallas_call(
        _mlp_body,
        grid=grid,
        in_specs=[
            pl.BlockSpec((block, h), row),
            pl.BlockSpec((block, h), row),
            pl.BlockSpec((block, 1), row),
            pl.BlockSpec((1, h), full),
            pl.BlockSpec((1, h), full),
            pl.BlockSpec((h, h), full),
            pl.BlockSpec((1, h), full),
        ],
        out_specs=pl.BlockSpec((block, h), row),
        out_shape=jax.ShapeDtypeStruct((e, h), jnp.float32),
    )(ga, gb, len2d, w1c, b1, w2, b2)


def kernel(scalars, edge_index, edge_len, W1, b1, W2, b2):
    n, d = scalars.shape
    h = W1.shape[1]
    wa = W1[:d]
    wb = W1[d:2 * d]
    w1c = W1[2 * d].reshape(1, h)
    pa, pb = _precompute(scalars, wa, wb)
    snd = edge_index[0]
    rcv = edge_index[1]
    ga, gb = _sc_gather(pa, pb, snd, rcv)
    return _edge_mlp(ga, gb, edge_len[:, None], w1c, b1.reshape(1, h), W2,
                     b2.reshape(1, h))


# speedup vs baseline: 3.1679x; 1.3008x over previous
import functools

import jax
import jax.numpy as jnp
from jax import lax
from jax.experimental import pallas as pl
from jax.experimental.pallas import tpu as pltpu
from jax.experimental.pallas import tpu_sc as plsc


def _precompute_body(scalars_ref, wa_ref, wb_ref, b1_ref, pa_ref, pb_ref):
    s = scalars_ref[...]
    pa_ref[...] = jnp.dot(s, wa_ref[...], preferred_element_type=jnp.float32)
    pb_ref[...] = (
        jnp.dot(s, wb_ref[...], preferred_element_type=jnp.float32)
        + b1_ref[...]
    )


def _precompute(scalars, wa, wb, b1):
    n, _ = scalars.shape
    h = wa.shape[1]
    return pl.pallas_call(
        _precompute_body,
        out_shape=[
            jax.ShapeDtypeStruct((n, h), jnp.float32),
            jax.ShapeDtypeStruct((n, h), jnp.float32),
        ],
    )(scalars, wa, wb, b1.reshape(1, h))


def _sc_gather(pa, pb, snd, rcv, chunk=400):
    n, h = pa.shape
    e = snd.shape[0]
    info = plsc.get_sparse_core_info()
    nc, ns = info.num_cores, info.num_subcores
    nw = nc * ns
    assert e % nw == 0
    epw = e // nw
    assert epw % chunk == 0 and chunk % 8 == 0
    steps = epw // chunk
    mesh = plsc.VectorSubcoreMesh(core_axis_name="c", subcore_axis_name="s")

    @functools.partial(
        pl.kernel,
        mesh=mesh,
        compiler_params=pltpu.CompilerParams(use_tc_tiling_on_sc=False),
        out_type=(
            jax.ShapeDtypeStruct((e, h), jnp.float32),
            jax.ShapeDtypeStruct((e, h), jnp.float32),
        ),
        scratch_types=[
            pltpu.VMEM((chunk,), jnp.int32),
            pltpu.VMEM((chunk,), jnp.int32),
            pltpu.VMEM((chunk, h), jnp.float32),
            pltpu.VMEM((chunk, h), jnp.float32),
            pltpu.SemaphoreType.DMA,
            pltpu.SemaphoreType.DMA,
        ],
    )
    def k(pa_hbm, pb_hbm, snd_hbm, rcv_hbm, ga_hbm, gb_hbm,
          idxa_v, idxb_v, bufa_v, bufb_v, sema, semb):
        wid = lax.axis_index("s") * nc + lax.axis_index("c")
        base0 = wid * epw

        def body(kk, carry):
            base = base0 + kk * chunk
            pltpu.sync_copy(snd_hbm.at[pl.ds(base, chunk)], idxa_v)
            pltpu.sync_copy(rcv_hbm.at[pl.ds(base, chunk)], idxb_v)
            cpa = pltpu.async_copy(pa_hbm.at[idxa_v], bufa_v, sema)
            cpb = pltpu.async_copy(pb_hbm.at[idxb_v], bufb_v, semb)
            cpa.wait()
            cpb.wait()
            pltpu.sync_copy(bufa_v, ga_hbm.at[pl.ds(base, chunk)])
            pltpu.sync_copy(bufb_v, gb_hbm.at[pl.ds(base, chunk)])
            return carry

        lax.fori_loop(0, steps, body, 0)

    return k(pa, pb, snd, rcv)


def _mlp_body(ga_ref, gb_ref, le_ref, lo_ref, w1c2_ref, w2d_ref, b2d_ref,
              out_ref):
    br, w = ga_ref.shape
    h = w // 2
    lane = lax.broadcasted_iota(jnp.int32, (br, w), 1)
    l2 = jnp.where(lane < h, le_ref[...], lo_ref[...])
    x = ga_ref[...] + gb_ref[...] + l2 * w1c2_ref[...]
    hh = x * jax.nn.sigmoid(x)
    out_ref[...] = (
        jnp.dot(hh, w2d_ref[...], preferred_element_type=jnp.float32)
        + b2d_ref[...]
    )


def _edge_mlp(ga2, gb2, le, lo, w1c2, w2d, b2d, block=1600):
    e2, w = ga2.shape
    h = w // 2
    assert e2 % block == 0
    grid = (e2 // block,)
    row = lambda i: (i, 0)
    full = lambda i: (0, 0)
    return pl.pallas_call(
        _mlp_body,
        grid=grid,
        in_specs=[
            pl.BlockSpec((block, w), row),
            pl.BlockSpec((block, w), row),
            pl.BlockSpec((block, 1), row),
            pl.BlockSpec((block, 1), row),
            pl.BlockSpec((1, w), full),
            pl.BlockSpec((w, w), full),
            pl.BlockSpec((1, w), full),
        ],
        out_specs=pl.BlockSpec((block, w), row),
        out_shape=jax.ShapeDtypeStruct((e2, w), jnp.float32),
    )(ga2, gb2, le, lo, w1c2, w2d, b2d)


def kernel(scalars, edge_index, edge_len, W1, b1, W2, b2):
    n, d = scalars.shape
    h = W1.shape[1]
    e = edge_index.shape[1]
    wa = W1[:d]
    wb = W1[d:2 * d]
    w1c = W1[2 * d].reshape(1, h)
    pa, pb = _precompute(scalars, wa, wb, b1)
    snd = edge_index[0]
    rcv = edge_index[1]
    ga, gb = _sc_gather(pa, pb, snd, rcv)
    ga2 = ga.reshape(e // 2, 2 * h)
    gb2 = gb.reshape(e // 2, 2 * h)
    le = edge_len[0::2].reshape(e // 2, 1)
    lo = edge_len[1::2].reshape(e // 2, 1)
    w1c2 = jnp.concatenate([w1c, w1c], axis=1)
    zero = jnp.zeros((h, h), jnp.float32)
    w2d = jnp.block([[W2, zero], [zero, W2]])
    b2d = jnp.concatenate([b2.reshape(1, h), b2.reshape(1, h)], axis=1)
    out2 = _edge_mlp(ga2, gb2, le, lo, w1c2, w2d, b2d)
    return out2.reshape(e, h)
